# Initial kernel scaffold; baseline (speedup 1.0000x reference)
import jax
import jax.numpy as jnp
from jax.experimental import pallas as pl


def kernel(x, edge_index, F1_W, F1_a_src, F1_a_dst, F1_b, F2_W, F2_a_src, F2_a_dst, F2_b, G1_W, G1_a_src, G1_a_dst, G1_b, G2_W, G2_a_src, G2_a_dst, G2_b):
    raise NotImplementedError("write your pallas kernel here")


# speedup vs baseline: 295.2255x; 295.2255x over previous
import functools

import jax
import jax.numpy as jnp
from jax import lax
from jax.experimental import pallas as pl
from jax.experimental.pallas import tpu as pltpu
from jax.experimental.pallas import tpu_sc as plsc

N = 100000
NPAD = 100096
DUMMY = N
E = 3200000
NTILES = 32
CHUNK = 256
SUB = CHUNK // 128
NCH = 392
EPT = CHUNK * NCH
EPAD = EPT * NTILES
RPT = NPAD // 16


def _table_body(x_ref, m_ref, o_ref):
    o_ref[...] = jnp.dot(x_ref[...], m_ref[...],
                         preferred_element_type=jnp.float32)


def _combine_body(a0_ref, a1_ref, x_ref, b_ref, o1_ref, o2_ref, ol_ref):
    a = a0_ref[...] + a1_ref[...]
    den = a[:, 0:4] + 1e-16
    b = b_ref[...]
    s1 = a[:, 4:6] / den[:, 0:1] + b[:, 0:2]
    t1 = a[:, 6:8] / den[:, 1:2] + b[:, 2:4]
    s2 = a[:, 8:10] / den[:, 2:3] + b[:, 4:6]
    t2 = a[:, 10:12] / den[:, 3:4] + b[:, 6:8]
    x2 = x_ref[:, 2:4]
    x1n = x2 * jnp.exp(s1) + t1
    x2n = x1n * jnp.exp(s2) + t2
    o1_ref[...] = x1n
    o2_ref[...] = x2n
    ol_ref[...] = jnp.sum(s1 + s2, axis=1, keepdims=True)


def _edge_body(table, src2, dst2, out,
               idx_s, idx_d, rows_s, rows_d, outbuf, acc, sem):
    c = lax.axis_index("c")
    s = lax.axis_index("s")
    wid = c * 16 + s

    zv = jnp.zeros((16,), jnp.float32)

    def _zero_outbuf(i, carry):
        outbuf[i, :] = zv
        return carry
    lax.fori_loop(0, CHUNK, _zero_outbuf, 0)

    def _zero_acc(i, carry):
        pltpu.sync_copy(outbuf, acc.at[pl.ds(s * RPT + i * CHUNK, CHUNK), :])
        return carry
    lax.fori_loop(0, RPT // CHUNK, _zero_acc, 0)
    pltpu.sync_copy(outbuf.at[pl.ds(0, RPT % CHUNK), :],
                    acc.at[pl.ds(s * RPT + (RPT // CHUNK) * CHUNK,
                                 RPT % CHUNK), :])
    plsc.subcore_barrier()

    lanes = lax.iota(jnp.int32, 16)
    row_base = wid * (EPT // 128)

    def _chunk(g, carry):
        r0 = row_base + g * SUB
        pltpu.sync_copy(src2.at[pl.ds(r0, SUB), :], idx_s)
        pltpu.sync_copy(dst2.at[pl.ds(r0, SUB), :], idx_d)
        cps = [pltpu.async_copy(table.at[idx_s.at[j]],
                                rows_s.at[pl.ds(j * 128, 128), :], sem)
               for j in range(SUB)]
        cps += [pltpu.async_copy(table.at[idx_d.at[j]],
                                 rows_d.at[pl.ds(j * 128, 128), :], sem)
                for j in range(SUB)]
        for cp in cps:
            cp.wait()

        for g2 in range(CHUNK // 16):
            ridx = lanes + (g2 * 16)

            def col(k):
                return jnp.full((16,), k, jnp.int32)

            aS = [plsc.load_gather(rows_s, [ridx, col(k)]) for k in range(4)]
            aD = [plsc.load_gather(rows_d, [ridx, col(4 + k)])
                  for k in range(4)]
            h = [plsc.load_gather(rows_s, [ridx, col(8 + k)])
                 for k in range(8)]
            for k in range(4):
                e = aS[k] + aD[k]
                e = jnp.maximum(e, 0.2 * e)
                p = jnp.exp(e)
                plsc.store_scatter(outbuf, [ridx, col(k)], p)
                plsc.store_scatter(outbuf, [ridx, col(4 + 2 * k)],
                                   h[2 * k] * p)
                plsc.store_scatter(outbuf, [ridx, col(5 + 2 * k)],
                                   h[2 * k + 1] * p)

        for j in range(SUB):
            pltpu.sync_copy(outbuf.at[pl.ds(j * 128, 128), :],
                            acc.at[idx_d.at[j]], add=True)
        return carry
    lax.fori_loop(0, NCH, _chunk, 0)
    plsc.subcore_barrier()

    pltpu.sync_copy(acc.at[pl.ds(s * RPT, RPT), :],
                    out.at[pl.ds(c * NPAD + s * RPT, RPT), :])


_edge_call = functools.partial(
    pl.kernel,
    mesh=plsc.VectorSubcoreMesh(core_axis_name="c", subcore_axis_name="s"),
    compiler_params=pltpu.CompilerParams(needs_layout_passes=False,
                                         use_tc_tiling_on_sc=False),
    out_type=jax.ShapeDtypeStruct((2 * NPAD, 16), jnp.float32),
    scratch_types=[
        pltpu.VMEM((SUB, 128), jnp.int32),
        pltpu.VMEM((SUB, 128), jnp.int32),
        pltpu.VMEM((CHUNK, 16), jnp.float32),
        pltpu.VMEM((CHUNK, 16), jnp.float32),
        pltpu.VMEM((CHUNK, 16), jnp.float32),
        pltpu.VMEM_SHARED((NPAD, 16), jnp.float32),
        pltpu.SemaphoreType.DMA,
    ],
)(_edge_body)


def _build_M(F1_W, F1_a_src, F1_a_dst, F2_W, F2_a_src, F2_a_dst,
             G1_W, G1_a_src, G1_a_dst, G2_W, G2_a_src, G2_a_dst):
    z = jnp.zeros((2,), jnp.float32)
    cols = []
    cols.append(jnp.concatenate([F1_W @ F1_a_src, z]))
    cols.append(jnp.concatenate([F2_W @ F2_a_src, z]))
    cols.append(jnp.concatenate([z, G1_W @ G1_a_src]))
    cols.append(jnp.concatenate([z, G2_W @ G2_a_src]))
    cols.append(jnp.concatenate([F1_W @ F1_a_dst, z]))
    cols.append(jnp.concatenate([F2_W @ F2_a_dst, z]))
    cols.append(jnp.concatenate([z, G1_W @ G1_a_dst]))
    cols.append(jnp.concatenate([z, G2_W @ G2_a_dst]))
    cols.append(jnp.concatenate([F1_W[:, 0], z]))
    cols.append(jnp.concatenate([F1_W[:, 1], z]))
    cols.append(jnp.concatenate([F2_W[:, 0], z]))
    cols.append(jnp.concatenate([F2_W[:, 1], z]))
    cols.append(jnp.concatenate([z, G1_W[:, 0]]))
    cols.append(jnp.concatenate([z, G1_W[:, 1]]))
    cols.append(jnp.concatenate([z, G2_W[:, 0]]))
    cols.append(jnp.concatenate([z, G2_W[:, 1]]))
    return jnp.stack(cols, axis=1)


def kernel(x, edge_index,
           F1_W, F1_a_src, F1_a_dst, F1_b,
           F2_W, F2_a_src, F2_a_dst, F2_b,
           G1_W, G1_a_src, G1_a_dst, G1_b,
           G2_W, G2_a_src, G2_a_dst, G2_b):
    M = _build_M(F1_W, F1_a_src, F1_a_dst, F2_W, F2_a_src, F2_a_dst,
                 G1_W, G1_a_src, G1_a_dst, G2_W, G2_a_src, G2_a_dst)
    x_pad = jnp.pad(x, ((0, NPAD - N), (0, 0)))

    TB = 6256
    table = pl.pallas_call(
        _table_body,
        grid=(NPAD // TB,),
        in_specs=[pl.BlockSpec((TB, 4), lambda i: (i, 0)),
                  pl.BlockSpec((4, 16), lambda i: (0, 0))],
        out_specs=pl.BlockSpec((TB, 16), lambda i: (i, 0)),
        out_shape=jax.ShapeDtypeStruct((NPAD, 16), jnp.float32),
    )(x_pad, M)

    pad_e = jnp.full((EPAD - E,), DUMMY, jnp.int32)
    src2 = jnp.concatenate([edge_index[0], pad_e]).reshape(EPAD // 128, 128)
    dst2 = jnp.concatenate([edge_index[1], pad_e]).reshape(EPAD // 128, 128)

    accf = _edge_call(table, src2, dst2)
    a0 = accf[:N]
    a1 = accf[NPAD:NPAD + N]

    b8 = jnp.concatenate([F1_b, F2_b, G1_b, G2_b]).reshape(1, 8)
    CB = 4000
    x1n, x2n, ld = pl.pallas_call(
        _combine_body,
        grid=(N // CB,),
        in_specs=[pl.BlockSpec((CB, 16), lambda i: (i, 0)),
                  pl.BlockSpec((CB, 16), lambda i: (i, 0)),
                  pl.BlockSpec((CB, 4), lambda i: (i, 0)),
                  pl.BlockSpec((1, 8), lambda i: (0, 0))],
        out_specs=[pl.BlockSpec((CB, 2), lambda i: (i, 0)),
                   pl.BlockSpec((CB, 2), lambda i: (i, 0)),
                   pl.BlockSpec((CB, 1), lambda i: (i, 0))],
        out_shape=[jax.ShapeDtypeStruct((N, 2), jnp.float32),
                   jax.ShapeDtypeStruct((N, 2), jnp.float32),
                   jax.ShapeDtypeStruct((N, 1), jnp.float32)],
    )(a0, a1, x, b8)

    return (x1n, x2n, ld.reshape(N))


# speedup vs baseline: 403.0468x; 1.3652x over previous
import functools

import jax
import jax.numpy as jnp
from jax import lax
from jax.experimental import pallas as pl
from jax.experimental.pallas import tpu as pltpu
from jax.experimental.pallas import tpu_sc as plsc

N = 100000
NPAD = 100096
DUMMY = N
E = 3200000
NTILES = 32
CHUNK = 128
NCH = 784
EPT = CHUNK * NCH
EPAD = EPT * NTILES
IDXB = 16
NBLK = NCH // IDXB
EALLOC = EPAD + IDXB * CHUNK
RPT = NPAD // 16


def _table_body(x_ref, m_ref, o_ref):
    o_ref[...] = jnp.dot(x_ref[...], m_ref[...],
                         preferred_element_type=jnp.float32)


def _combine_body(a0_ref, a1_ref, x_ref, b_ref, o1_ref, o2_ref, ol_ref):
    a = a0_ref[...] + a1_ref[...]
    den = a[:, 0:4] + 1e-16
    b = b_ref[...]
    s1 = a[:, 4:6] / den[:, 0:1] + b[:, 0:2]
    t1 = a[:, 6:8] / den[:, 1:2] + b[:, 2:4]
    s2 = a[:, 8:10] / den[:, 2:3] + b[:, 4:6]
    t2 = a[:, 10:12] / den[:, 3:4] + b[:, 6:8]
    x2 = x_ref[:, 2:4]
    x1n = x2 * jnp.exp(s1) + t1
    x2n = x1n * jnp.exp(s2) + t2
    o1_ref[...] = x1n
    o2_ref[...] = x2n
    ol_ref[...] = jnp.sum(s1 + s2, axis=1, keepdims=True)


def _edge_body(table, src2, dst2, out,
               idx_s, idx_d, rows_s, rows_d, outbuf, acc,
               gsem, ssem, isem):
    c = lax.axis_index("c")
    s = lax.axis_index("s")
    wid = c * 16 + s

    zv = jnp.zeros((16,), jnp.float32)

    def _zero_outbuf(i, carry):
        outbuf[i, :] = zv
        return carry
    lax.fori_loop(0, 2 * CHUNK, _zero_outbuf, 0)

    def _zero_acc(i, carry):
        pltpu.sync_copy(outbuf,
                        acc.at[pl.ds(s * RPT + i * 2 * CHUNK, 2 * CHUNK), :])
        return carry
    lax.fori_loop(0, RPT // (2 * CHUNK), _zero_acc, 0)
    pltpu.sync_copy(outbuf.at[pl.ds(0, RPT % (2 * CHUNK)), :],
                    acc.at[pl.ds(s * RPT + (RPT // (2 * CHUNK)) * 2 * CHUNK,
                                 RPT % (2 * CHUNK)), :])
    plsc.subcore_barrier()

    lanes = lax.iota(jnp.int32, 16)
    row_base = wid * NCH

    def _issue_idx_block(blk):
        slot = lax.rem(blk, 3) * IDXB
        hrow = row_base + blk * IDXB
        pltpu.async_copy(src2.at[pl.ds(hrow, IDXB), :],
                         idx_s.at[pl.ds(slot, IDXB), :], isem)
        pltpu.async_copy(dst2.at[pl.ds(hrow, IDXB), :],
                         idx_d.at[pl.ds(slot, IDXB), :], isem)

    def _wait_idx_block():
        for _ in range(2):
            pltpu.make_async_copy(src2.at[pl.ds(0, IDXB), :],
                                  idx_s.at[pl.ds(0, IDXB), :], isem).wait()

    def _issue_gathers(g, q):
        row = lax.rem(g // IDXB, 3) * IDXB + lax.rem(g, IDXB)
        pltpu.async_copy(table.at[idx_s.at[row]],
                         rows_s.at[pl.ds(q * CHUNK, CHUNK), :], gsem)
        pltpu.async_copy(table.at[idx_d.at[row]],
                         rows_d.at[pl.ds(q * CHUNK, CHUNK), :], gsem)

    def _wait_gathers():
        for buf in (rows_s, rows_d):
            pltpu.make_async_copy(table.at[idx_s.at[0]],
                                  buf.at[pl.ds(0, CHUNK), :], gsem).wait()

    pltpu.sync_copy(src2.at[pl.ds(row_base, IDXB), :],
                    idx_s.at[pl.ds(0, IDXB), :])
    pltpu.sync_copy(dst2.at[pl.ds(row_base, IDXB), :],
                    idx_d.at[pl.ds(0, IDXB), :])
    _issue_idx_block(jnp.int32(1))
    _issue_gathers(jnp.int32(0), jnp.int32(0))

    def _chunk(g, carry):
        q = lax.rem(g, 2)
        blk = g // IDXB
        off = lax.rem(g, IDXB)
        row = lax.rem(blk, 3) * IDXB + off

        @pl.when(jnp.logical_and(off == 0, blk >= 1))
        def _():
            _issue_idx_block(blk + 1)

        _wait_gathers()

        @pl.when(g >= 1)
        def _():
            pltpu.make_async_copy(
                outbuf.at[pl.ds(0, CHUNK), :],
                acc.at[idx_d.at[0]], ssem).wait()

        @pl.when(off == IDXB - 1)
        def _():
            _wait_idx_block()

        _issue_gathers(g + 1, 1 - q)

        qr = q * CHUNK
        for g2 in range(CHUNK // 16):
            ridx = lanes + (g2 * 16 + qr)

            def col(k):
                return jnp.full((16,), k, jnp.int32)

            aS = [plsc.load_gather(rows_s, [ridx, col(k)]) for k in range(4)]
            aD = [plsc.load_gather(rows_d, [ridx, col(4 + k)])
                  for k in range(4)]
            h = [plsc.load_gather(rows_s, [ridx, col(8 + k)])
                 for k in range(8)]
            for k in range(4):
                e = aS[k] + aD[k]
                e = jnp.maximum(e, 0.2 * e)
                p = jnp.exp(e)
                plsc.store_scatter(outbuf, [ridx, col(k)], p)
                plsc.store_scatter(outbuf, [ridx, col(4 + 2 * k)],
                                   h[2 * k] * p)
                plsc.store_scatter(outbuf, [ridx, col(5 + 2 * k)],
                                   h[2 * k + 1] * p)

        pltpu.async_copy(outbuf.at[pl.ds(qr, CHUNK), :],
                         acc.at[idx_d.at[row]], ssem, add=True)
        return carry
    lax.fori_loop(0, NCH, _chunk, 0)

    pltpu.make_async_copy(outbuf.at[pl.ds(0, CHUNK), :],
                          acc.at[idx_d.at[0]], ssem).wait()
    _wait_gathers()
    plsc.subcore_barrier()

    pltpu.sync_copy(acc.at[pl.ds(s * RPT, RPT), :],
                    out.at[pl.ds(c * NPAD + s * RPT, RPT), :])


_edge_call = functools.partial(
    pl.kernel,
    mesh=plsc.VectorSubcoreMesh(core_axis_name="c", subcore_axis_name="s"),
    compiler_params=pltpu.CompilerParams(needs_layout_passes=False,
                                         use_tc_tiling_on_sc=False),
    out_type=jax.ShapeDtypeStruct((2 * NPAD, 16), jnp.float32),
    scratch_types=[
        pltpu.VMEM((3 * IDXB, 128), jnp.int32),
        pltpu.VMEM((3 * IDXB, 128), jnp.int32),
        pltpu.VMEM((2 * CHUNK, 16), jnp.float32),
        pltpu.VMEM((2 * CHUNK, 16), jnp.float32),
        pltpu.VMEM((2 * CHUNK, 16), jnp.float32),
        pltpu.VMEM_SHARED((NPAD, 16), jnp.float32),
        pltpu.SemaphoreType.DMA,
        pltpu.SemaphoreType.DMA,
        pltpu.SemaphoreType.DMA,
    ],
)(_edge_body)


def _build_M(F1_W, F1_a_src, F1_a_dst, F2_W, F2_a_src, F2_a_dst,
             G1_W, G1_a_src, G1_a_dst, G2_W, G2_a_src, G2_a_dst):
    z = jnp.zeros((2,), jnp.float32)
    cols = []
    cols.append(jnp.concatenate([F1_W @ F1_a_src, z]))
    cols.append(jnp.concatenate([F2_W @ F2_a_src, z]))
    cols.append(jnp.concatenate([z, G1_W @ G1_a_src]))
    cols.append(jnp.concatenate([z, G2_W @ G2_a_src]))
    cols.append(jnp.concatenate([F1_W @ F1_a_dst, z]))
    cols.append(jnp.concatenate([F2_W @ F2_a_dst, z]))
    cols.append(jnp.concatenate([z, G1_W @ G1_a_dst]))
    cols.append(jnp.concatenate([z, G2_W @ G2_a_dst]))
    cols.append(jnp.concatenate([F1_W[:, 0], z]))
    cols.append(jnp.concatenate([F1_W[:, 1], z]))
    cols.append(jnp.concatenate([F2_W[:, 0], z]))
    cols.append(jnp.concatenate([F2_W[:, 1], z]))
    cols.append(jnp.concatenate([z, G1_W[:, 0]]))
    cols.append(jnp.concatenate([z, G1_W[:, 1]]))
    cols.append(jnp.concatenate([z, G2_W[:, 0]]))
    cols.append(jnp.concatenate([z, G2_W[:, 1]]))
    return jnp.stack(cols, axis=1)


def kernel(x, edge_index,
           F1_W, F1_a_src, F1_a_dst, F1_b,
           F2_W, F2_a_src, F2_a_dst, F2_b,
           G1_W, G1_a_src, G1_a_dst, G1_b,
           G2_W, G2_a_src, G2_a_dst, G2_b):
    M = _build_M(F1_W, F1_a_src, F1_a_dst, F2_W, F2_a_src, F2_a_dst,
                 G1_W, G1_a_src, G1_a_dst, G2_W, G2_a_src, G2_a_dst)
    x_pad = jnp.pad(x, ((0, NPAD - N), (0, 0)))

    TB = 6256
    table = pl.pallas_call(
        _table_body,
        grid=(NPAD // TB,),
        in_specs=[pl.BlockSpec((TB, 4), lambda i: (i, 0)),
                  pl.BlockSpec((4, 16), lambda i: (0, 0))],
        out_specs=pl.BlockSpec((TB, 16), lambda i: (i, 0)),
        out_shape=jax.ShapeDtypeStruct((NPAD, 16), jnp.float32),
    )(x_pad, M)

    pad_e = jnp.full((EALLOC - E,), DUMMY, jnp.int32)
    src2 = jnp.concatenate([edge_index[0], pad_e]).reshape(EALLOC // 128, 128)
    dst2 = jnp.concatenate([edge_index[1], pad_e]).reshape(EALLOC // 128, 128)

    accf = _edge_call(table, src2, dst2)
    a0 = accf[:N]
    a1 = accf[NPAD:NPAD + N]

    b8 = jnp.concatenate([F1_b, F2_b, G1_b, G2_b]).reshape(1, 8)
    CB = 4000
    x1n, x2n, ld = pl.pallas_call(
        _combine_body,
        grid=(N // CB,),
        in_specs=[pl.BlockSpec((CB, 16), lambda i: (i, 0)),
                  pl.BlockSpec((CB, 16), lambda i: (i, 0)),
                  pl.BlockSpec((CB, 4), lambda i: (i, 0)),
                  pl.BlockSpec((1, 8), lambda i: (0, 0))],
        out_specs=[pl.BlockSpec((CB, 2), lambda i: (i, 0)),
                   pl.BlockSpec((CB, 2), lambda i: (i, 0)),
                   pl.BlockSpec((CB, 1), lambda i: (i, 0))],
        out_shape=[jax.ShapeDtypeStruct((N, 2), jnp.float32),
                   jax.ShapeDtypeStruct((N, 2), jnp.float32),
                   jax.ShapeDtypeStruct((N, 1), jnp.float32)],
    )(a0, a1, x, b8)

    return (x1n, x2n, ld.reshape(N))


# speedup vs baseline: 543.9628x; 1.3496x over previous
import functools

import jax
import jax.numpy as jnp
from jax import lax
from jax.experimental import pallas as pl
from jax.experimental.pallas import tpu as pltpu
from jax.experimental.pallas import tpu_sc as plsc

N = 100000
NPAD = 100096
DUMMY = N
E = 3200000
NTILES = 32
CHUNK = 128
NCH = 784
EPT = CHUNK * NCH
EPAD = EPT * NTILES
IDXB = 8
NBLK = NCH // IDXB
EALLOC = EPAD + IDXB * CHUNK
RPT = NPAD // 16


def _table_body(x_ref, m_ref, o_ref):
    o_ref[...] = jnp.dot(x_ref[...], m_ref[...],
                         preferred_element_type=jnp.float32)


def _combine_body(a0_ref, a1_ref, x_ref, b_ref, o1_ref, o2_ref, ol_ref):
    a = a0_ref[...] + a1_ref[...]
    den = a[:, 0:4] + 1e-16
    b = b_ref[...]
    s1 = a[:, 4:6] / den[:, 0:1] + b[:, 0:2]
    t1 = a[:, 6:8] / den[:, 1:2] + b[:, 2:4]
    s2 = a[:, 8:10] / den[:, 2:3] + b[:, 4:6]
    t2 = a[:, 10:12] / den[:, 3:4] + b[:, 6:8]
    x2 = x_ref[:, 2:4]
    x1n = x2 * jnp.exp(s1) + t1
    x2n = x1n * jnp.exp(s2) + t2
    o1_ref[...] = x1n
    o2_ref[...] = x2n
    ol_ref[...] = jnp.sum(s1 + s2, axis=1, keepdims=True)


def _edge_body(table, src2, dst2, out,
               idx_s, idx_d, rows_s, rows_d, outbuf, acc,
               gsem0, gsem1, gsem2, gsem3, ssem0, ssem1, isem):
    gsems = (gsem0, gsem1, gsem2, gsem3)
    ssems = (ssem0, ssem1)
    c = lax.axis_index("c")
    s = lax.axis_index("s")
    wid = c * 16 + s

    zv = jnp.zeros((16,), jnp.float32)

    def _zero_outbuf(i, carry):
        outbuf[i, :] = zv
        return carry
    lax.fori_loop(0, 2 * CHUNK, _zero_outbuf, 0)

    def _zero_acc(i, carry):
        pltpu.sync_copy(outbuf,
                        acc.at[pl.ds(s * RPT + i * 2 * CHUNK, 2 * CHUNK), :])
        return carry
    lax.fori_loop(0, RPT // (2 * CHUNK), _zero_acc, 0)
    pltpu.sync_copy(outbuf.at[pl.ds(0, RPT % (2 * CHUNK)), :],
                    acc.at[pl.ds(s * RPT + (RPT // (2 * CHUNK)) * 2 * CHUNK,
                                 RPT % (2 * CHUNK)), :])
    plsc.subcore_barrier()

    lanes = lax.iota(jnp.int32, 16)
    row_base = wid * NCH

    def _issue_idx_block(blk):
        slot = lax.rem(blk, 3) * IDXB
        hrow = row_base + blk * IDXB
        pltpu.async_copy(src2.at[pl.ds(hrow, IDXB), :],
                         idx_s.at[pl.ds(slot, IDXB), :], isem)
        pltpu.async_copy(dst2.at[pl.ds(hrow, IDXB), :],
                         idx_d.at[pl.ds(slot, IDXB), :], isem)

    def _wait_idx_block():
        for _ in range(2):
            pltpu.make_async_copy(src2.at[pl.ds(0, IDXB), :],
                                  idx_s.at[pl.ds(0, IDXB), :], isem).wait()

    def _idx_row(t):
        return lax.rem(t // IDXB, 3) * IDXB + lax.rem(t, IDXB)

    def _issue_gathers(t, slot, sem):
        row = _idx_row(t)
        pltpu.async_copy(table.at[idx_s.at[row]],
                         rows_s.at[pl.ds(slot * CHUNK, CHUNK), :], sem)
        pltpu.async_copy(table.at[idx_d.at[row]],
                         rows_d.at[pl.ds(slot * CHUNK, CHUNK), :], sem)

    def _wait_gathers(sem):
        for buf in (rows_s, rows_d):
            pltpu.make_async_copy(table.at[idx_s.at[0]],
                                  buf.at[pl.ds(0, CHUNK), :], sem).wait()

    def _wait_scatter(sem):
        pltpu.make_async_copy(outbuf.at[pl.ds(0, CHUNK), :],
                              acc.at[idx_d.at[0]], sem).wait()

    pltpu.sync_copy(src2.at[pl.ds(row_base, IDXB), :],
                    idx_s.at[pl.ds(0, IDXB), :])
    pltpu.sync_copy(dst2.at[pl.ds(row_base, IDXB), :],
                    idx_d.at[pl.ds(0, IDXB), :])
    _issue_idx_block(jnp.int32(1))
    for t in range(3):
        _issue_gathers(jnp.int32(t), t, gsems[t])

    def _macro(i, carry):
        for j in range(4):
            g = 4 * i + j
            if j == 0:
                blk = i // 2

                @pl.when(jnp.logical_and(lax.rem(i, 2) == 0, i >= 2))
                def _():
                    _issue_idx_block(blk + 1)

                @pl.when(lax.rem(i, 2) == 1)
                def _():
                    _wait_idx_block()

            _wait_gathers(gsems[j])

            if j >= 2:
                _wait_scatter(ssems[j % 2])
            else:
                @pl.when(i >= 1)
                def _():
                    _wait_scatter(ssems[j % 2])

            _issue_gathers(g + 3, (j + 3) % 4, gsems[(j + 3) % 4])

            qr = (j % 2) * CHUNK
            base = j * CHUNK
            for g2 in range(CHUNK // 16):
                ridx = lanes + (g2 * 16 + base)
                oidx = lanes + (g2 * 16 + qr)

                def col(k):
                    return jnp.full((16,), k, jnp.int32)

                aS = [plsc.load_gather(rows_s, [ridx, col(k)])
                      for k in range(4)]
                aD = [plsc.load_gather(rows_d, [ridx, col(4 + k)])
                      for k in range(4)]
                h = [plsc.load_gather(rows_s, [ridx, col(8 + k)])
                     for k in range(8)]
                for k in range(4):
                    e = aS[k] + aD[k]
                    e = jnp.maximum(e, 0.2 * e)
                    p = jnp.exp(e)
                    plsc.store_scatter(outbuf, [oidx, col(k)], p)
                    plsc.store_scatter(outbuf, [oidx, col(4 + 2 * k)],
                                       h[2 * k] * p)
                    plsc.store_scatter(outbuf, [oidx, col(5 + 2 * k)],
                                       h[2 * k + 1] * p)

            pltpu.async_copy(outbuf.at[pl.ds(qr, CHUNK), :],
                             acc.at[idx_d.at[_idx_row(g)]],
                             ssems[j % 2], add=True)
        return carry
    lax.fori_loop(0, NCH // 4, _macro, 0)

    _wait_scatter(ssems[0])
    _wait_scatter(ssems[1])
    for t in range(3):
        _wait_gathers(gsems[t])
    plsc.subcore_barrier()

    pltpu.sync_copy(acc.at[pl.ds(s * RPT, RPT), :],
                    out.at[pl.ds(c * NPAD + s * RPT, RPT), :])


_edge_call = functools.partial(
    pl.kernel,
    mesh=plsc.VectorSubcoreMesh(core_axis_name="c", subcore_axis_name="s"),
    compiler_params=pltpu.CompilerParams(needs_layout_passes=False,
                                         use_tc_tiling_on_sc=False),
    out_type=jax.ShapeDtypeStruct((2 * NPAD, 16), jnp.float32),
    scratch_types=[
        pltpu.VMEM((3 * IDXB, 128), jnp.int32),
        pltpu.VMEM((3 * IDXB, 128), jnp.int32),
        pltpu.VMEM((4 * CHUNK, 16), jnp.float32),
        pltpu.VMEM((4 * CHUNK, 16), jnp.float32),
        pltpu.VMEM((2 * CHUNK, 16), jnp.float32),
        pltpu.VMEM_SHARED((NPAD, 16), jnp.float32),
        pltpu.SemaphoreType.DMA,
        pltpu.SemaphoreType.DMA,
        pltpu.SemaphoreType.DMA,
        pltpu.SemaphoreType.DMA,
        pltpu.SemaphoreType.DMA,
        pltpu.SemaphoreType.DMA,
        pltpu.SemaphoreType.DMA,
    ],
)(_edge_body)


def _build_M(F1_W, F1_a_src, F1_a_dst, F2_W, F2_a_src, F2_a_dst,
             G1_W, G1_a_src, G1_a_dst, G2_W, G2_a_src, G2_a_dst):
    z = jnp.zeros((2,), jnp.float32)
    cols = []
    cols.append(jnp.concatenate([F1_W @ F1_a_src, z]))
    cols.append(jnp.concatenate([F2_W @ F2_a_src, z]))
    cols.append(jnp.concatenate([z, G1_W @ G1_a_src]))
    cols.append(jnp.concatenate([z, G2_W @ G2_a_src]))
    cols.append(jnp.concatenate([F1_W @ F1_a_dst, z]))
    cols.append(jnp.concatenate([F2_W @ F2_a_dst, z]))
    cols.append(jnp.concatenate([z, G1_W @ G1_a_dst]))
    cols.append(jnp.concatenate([z, G2_W @ G2_a_dst]))
    cols.append(jnp.concatenate([F1_W[:, 0], z]))
    cols.append(jnp.concatenate([F1_W[:, 1], z]))
    cols.append(jnp.concatenate([F2_W[:, 0], z]))
    cols.append(jnp.concatenate([F2_W[:, 1], z]))
    cols.append(jnp.concatenate([z, G1_W[:, 0]]))
    cols.append(jnp.concatenate([z, G1_W[:, 1]]))
    cols.append(jnp.concatenate([z, G2_W[:, 0]]))
    cols.append(jnp.concatenate([z, G2_W[:, 1]]))
    return jnp.stack(cols, axis=1)


def kernel(x, edge_index,
           F1_W, F1_a_src, F1_a_dst, F1_b,
           F2_W, F2_a_src, F2_a_dst, F2_b,
           G1_W, G1_a_src, G1_a_dst, G1_b,
           G2_W, G2_a_src, G2_a_dst, G2_b):
    M = _build_M(F1_W, F1_a_src, F1_a_dst, F2_W, F2_a_src, F2_a_dst,
                 G1_W, G1_a_src, G1_a_dst, G2_W, G2_a_src, G2_a_dst)
    x_pad = jnp.pad(x, ((0, NPAD - N), (0, 0)))

    TB = 6256
    table = pl.pallas_call(
        _table_body,
        grid=(NPAD // TB,),
        in_specs=[pl.BlockSpec((TB, 4), lambda i: (i, 0)),
                  pl.BlockSpec((4, 16), lambda i: (0, 0))],
        out_specs=pl.BlockSpec((TB, 16), lambda i: (i, 0)),
        out_shape=jax.ShapeDtypeStruct((NPAD, 16), jnp.float32),
    )(x_pad, M)

    pad_e = jnp.full((EALLOC - E,), DUMMY, jnp.int32)
    src2 = jnp.concatenate([edge_index[0], pad_e]).reshape(EALLOC // 128, 128)
    dst2 = jnp.concatenate([edge_index[1], pad_e]).reshape(EALLOC // 128, 128)

    accf = _edge_call(table, src2, dst2)
    a0 = accf[:N]
    a1 = accf[NPAD:NPAD + N]

    b8 = jnp.concatenate([F1_b, F2_b, G1_b, G2_b]).reshape(1, 8)
    CB = 4000
    x1n, x2n, ld = pl.pallas_call(
        _combine_body,
        grid=(N // CB,),
        in_specs=[pl.BlockSpec((CB, 16), lambda i: (i, 0)),
                  pl.BlockSpec((CB, 16), lambda i: (i, 0)),
                  pl.BlockSpec((CB, 4), lambda i: (i, 0)),
                  pl.BlockSpec((1, 8), lambda i: (0, 0))],
        out_specs=[pl.BlockSpec((CB, 2), lambda i: (i, 0)),
                   pl.BlockSpec((CB, 2), lambda i: (i, 0)),
                   pl.BlockSpec((CB, 1), lambda i: (i, 0))],
        out_shape=[jax.ShapeDtypeStruct((N, 2), jnp.float32),
                   jax.ShapeDtypeStruct((N, 2), jnp.float32),
                   jax.ShapeDtypeStruct((N, 1), jnp.float32)],
    )(a0, a1, x, b8)

    return (x1n, x2n, ld.reshape(N))


# speedup vs baseline: 674.7220x; 1.2404x over previous
import functools

import jax
import jax.numpy as jnp
from jax import lax
from jax.experimental import pallas as pl
from jax.experimental.pallas import tpu as pltpu
from jax.experimental.pallas import tpu_sc as plsc

N = 100000
NPAD = 100352
DUMMY = N
E = 3200000
NTILES = 32
CHUNK = 128
NCH = 784
EPT = CHUNK * NCH
EPAD = EPT * NTILES
IDXB = 8
NBLK = NCH // IDXB
EALLOC = EPAD + IDXB * CHUNK
NT = 100096
RPT = NT // 16
TCH = 272
CROWS = NPAD // 32
CCH = 784


def _col(k):
    return jnp.full((16,), k, jnp.int32)


def _edge_body(x_pad, m16, src2, dst2, out, tbl,
               idx_s, idx_d, rows_s, rows_d, outbuf, xstage, msc, acc,
               gsem0, gsem1, gsem2, gsem3, ssem0, ssem1, isem):
    gsems = (gsem0, gsem1, gsem2, gsem3)
    ssems = (ssem0, ssem1)
    c = lax.axis_index("c")
    s = lax.axis_index("s")
    wid = c * 16 + s
    lanes = lax.iota(jnp.int32, 16)

    zv = jnp.zeros((16,), jnp.float32)

    def _zero_outbuf(i, carry):
        outbuf[i, :] = zv
        return carry
    lax.fori_loop(0, 2 * CHUNK, _zero_outbuf, 0)

    def _zero_acc(i, carry):
        pltpu.sync_copy(outbuf,
                        acc.at[pl.ds(s * RPT + i * 2 * CHUNK, 2 * CHUNK), :])
        return carry
    lax.fori_loop(0, RPT // (2 * CHUNK), _zero_acc, 0)
    pltpu.sync_copy(outbuf.at[pl.ds(0, RPT % (2 * CHUNK)), :],
                    acc.at[pl.ds(s * RPT + (RPT // (2 * CHUNK)) * 2 * CHUNK,
                                 RPT % (2 * CHUNK)), :])

    pltpu.sync_copy(m16, msc)
    tb = s * RPT

    def _tbl_chunk(i, carry):
        r0 = tb + i * TCH
        pltpu.sync_copy(x_pad.at[pl.ds(r0, TCH), :], xstage)

        def _tbl_group(gi, carry2):
            ridx = lanes + gi * 16
            mrows = [msc[j, :] for j in range(4)]
            xv = [plsc.load_gather(xstage, [ridx, _col(j)]) for j in range(4)]
            for ch in range(16):
                t = xv[0] * mrows[0][ch]
                for j in range(1, 4):
                    t = t + xv[j] * mrows[j][ch]
                plsc.store_scatter(rows_s, [ridx, _col(ch)], t)
            return carry2
        lax.fori_loop(0, TCH // 16, _tbl_group, 0)
        pltpu.sync_copy(rows_s.at[pl.ds(0, TCH), :],
                        tbl.at[c].at[pl.ds(r0, TCH), :])
        return carry
    lax.fori_loop(0, RPT // TCH, _tbl_chunk, 0)
    plsc.subcore_barrier()

    table = tbl.at[c]
    row_base = wid * NCH

    def _issue_idx_block(blk):
        slot = lax.rem(blk, 3) * IDXB
        hrow = row_base + blk * IDXB
        pltpu.async_copy(src2.at[pl.ds(hrow, IDXB), :],
                         idx_s.at[pl.ds(slot, IDXB), :], isem)
        pltpu.async_copy(dst2.at[pl.ds(hrow, IDXB), :],
                         idx_d.at[pl.ds(slot, IDXB), :], isem)

    def _wait_idx_block():
        for _ in range(2):
            pltpu.make_async_copy(src2.at[pl.ds(0, IDXB), :],
                                  idx_s.at[pl.ds(0, IDXB), :], isem).wait()

    def _idx_row(t):
        return lax.rem(t // IDXB, 3) * IDXB + lax.rem(t, IDXB)

    def _issue_gathers(t, slot, sem):
        row = _idx_row(t)
        pltpu.async_copy(table.at[idx_s.at[row]],
                         rows_s.at[pl.ds(slot * CHUNK, CHUNK), :], sem)
        pltpu.async_copy(table.at[idx_d.at[row]],
                         rows_d.at[pl.ds(slot * CHUNK, CHUNK), :], sem)

    def _wait_gathers(sem):
        for buf in (rows_s, rows_d):
            pltpu.make_async_copy(table.at[idx_s.at[0]],
                                  buf.at[pl.ds(0, CHUNK), :], sem).wait()

    def _wait_scatter(sem):
        pltpu.make_async_copy(outbuf.at[pl.ds(0, CHUNK), :],
                              acc.at[idx_d.at[0]], sem).wait()

    pltpu.sync_copy(src2.at[pl.ds(row_base, IDXB), :],
                    idx_s.at[pl.ds(0, IDXB), :])
    pltpu.sync_copy(dst2.at[pl.ds(row_base, IDXB), :],
                    idx_d.at[pl.ds(0, IDXB), :])
    _issue_idx_block(jnp.int32(1))
    for t in range(3):
        _issue_gathers(jnp.int32(t), t, gsems[t])

    def _macro(i, carry):
        for j in range(4):
            g = 4 * i + j
            if j == 0:
                blk = i // 2

                @pl.when(jnp.logical_and(lax.rem(i, 2) == 0, i >= 2))
                def _():
                    _issue_idx_block(blk + 1)

                @pl.when(lax.rem(i, 2) == 1)
                def _():
                    _wait_idx_block()

            _wait_gathers(gsems[j])

            if j >= 2:
                _wait_scatter(ssems[j % 2])
            else:
                @pl.when(i >= 1)
                def _():
                    _wait_scatter(ssems[j % 2])

            _issue_gathers(g + 3, (j + 3) % 4, gsems[(j + 3) % 4])

            qr = (j % 2) * CHUNK
            base = j * CHUNK

            def _group(g2, carry2):
                ridx = lanes + (g2 * 16 + base)
                oidx = lanes + (g2 * 16 + qr)
                aS = [plsc.load_gather(rows_s, [ridx, _col(k)])
                      for k in range(4)]
                aD = [plsc.load_gather(rows_d, [ridx, _col(4 + k)])
                      for k in range(4)]
                h = [plsc.load_gather(rows_s, [ridx, _col(8 + k)])
                     for k in range(8)]
                for k in range(4):
                    e = aS[k] + aD[k]
                    e = jnp.maximum(e, 0.2 * e)
                    p = jnp.exp(e)
                    plsc.store_scatter(outbuf, [oidx, _col(k)], p)
                    plsc.store_scatter(outbuf, [oidx, _col(4 + 2 * k)],
                                       h[2 * k] * p)
                    plsc.store_scatter(outbuf, [oidx, _col(5 + 2 * k)],
                                       h[2 * k + 1] * p)
                return carry2
            lax.fori_loop(0, CHUNK // 16, _group, 0)

            pltpu.async_copy(outbuf.at[pl.ds(qr, CHUNK), :],
                             acc.at[idx_d.at[_idx_row(g)]],
                             ssems[j % 2], add=True)
        return carry
    lax.fori_loop(0, NCH // 4, _macro, 0)

    _wait_scatter(ssems[0])
    _wait_scatter(ssems[1])
    for t in range(3):
        _wait_gathers(gsems[t])
    plsc.subcore_barrier()

    pltpu.sync_copy(acc.at[pl.ds(s * RPT, RPT), :],
                    out.at[pl.ds(c * NPAD + s * RPT, RPT), :])


_edge_call = functools.partial(
    pl.kernel,
    mesh=plsc.VectorSubcoreMesh(core_axis_name="c", subcore_axis_name="s"),
    compiler_params=pltpu.CompilerParams(needs_layout_passes=False,
                                         use_tc_tiling_on_sc=False,
                                         internal_scratch_in_bytes=32768),
    out_type=[jax.ShapeDtypeStruct((2 * NPAD, 16), jnp.float32),
              jax.ShapeDtypeStruct((2, NT, 16), jnp.float32)],
    scratch_types=[
        pltpu.VMEM((3 * IDXB, 128), jnp.int32),
        pltpu.VMEM((3 * IDXB, 128), jnp.int32),
        pltpu.VMEM((4 * CHUNK, 16), jnp.float32),
        pltpu.VMEM((4 * CHUNK, 16), jnp.float32),
        pltpu.VMEM((2 * CHUNK, 16), jnp.float32),
        pltpu.VMEM((TCH, 4), jnp.float32),
        pltpu.VMEM((4, 16), jnp.float32),
        pltpu.VMEM_SHARED((NT, 16), jnp.float32),
        pltpu.SemaphoreType.DMA,
        pltpu.SemaphoreType.DMA,
        pltpu.SemaphoreType.DMA,
        pltpu.SemaphoreType.DMA,
        pltpu.SemaphoreType.DMA,
        pltpu.SemaphoreType.DMA,
        pltpu.SemaphoreType.DMA,
    ],
)(_edge_body)


def _combine_body(accf, x_pad, b8, y1, y2, ld,
                  a0s, a1s, xs, y1s, y2s, lds, bsc):
    c = lax.axis_index("c")
    s = lax.axis_index("s")
    wid = c * 16 + s
    base = wid * CROWS
    lanes = lax.iota(jnp.int32, 16)

    pltpu.sync_copy(b8, bsc)
    _brow = bsc[...]
    bv = [_brow[k] for k in range(8)]

    def _chunk(i, carry):
        r0 = base + i * CCH
        pltpu.sync_copy(accf.at[pl.ds(r0, CCH), :], a0s)
        pltpu.sync_copy(accf.at[pl.ds(NPAD + r0, CCH), :], a1s)
        pltpu.sync_copy(x_pad.at[pl.ds(r0, CCH), :], xs)

        def _group(gi, carry2):
            ridx = lanes + gi * 16
            oidx = ridx + i * CCH
            a = [plsc.load_gather(a0s, [ridx, _col(k)])
                 + plsc.load_gather(a1s, [ridx, _col(k)])
                 for k in range(12)]
            den = [a[k] + 1e-16 for k in range(4)]
            s10 = a[4] / den[0] + bv[0]
            s11 = a[5] / den[0] + bv[1]
            t10 = a[6] / den[1] + bv[2]
            t11 = a[7] / den[1] + bv[3]
            s20 = a[8] / den[2] + bv[4]
            s21 = a[9] / den[2] + bv[5]
            t20 = a[10] / den[3] + bv[6]
            t21 = a[11] / den[3] + bv[7]
            x20 = plsc.load_gather(xs, [ridx, _col(2)])
            x21 = plsc.load_gather(xs, [ridx, _col(3)])
            x1n0 = x20 * jnp.exp(s10) + t10
            x1n1 = x21 * jnp.exp(s11) + t11
            x2n0 = x1n0 * jnp.exp(s20) + t20
            x2n1 = x1n1 * jnp.exp(s21) + t21
            plsc.store_scatter(y1s, [oidx, _col(0)], x1n0)
            plsc.store_scatter(y1s, [oidx, _col(1)], x1n1)
            plsc.store_scatter(y2s, [oidx, _col(0)], x2n0)
            plsc.store_scatter(y2s, [oidx, _col(1)], x2n1)
            lds[pl.ds(i * CCH + gi * 16, 16)] = s10 + s11 + s20 + s21
            return carry2
        lax.fori_loop(0, CCH // 16, _group, 0)
        return carry
    lax.fori_loop(0, CROWS // CCH, _chunk, 0)

    pltpu.sync_copy(y1s, y1.at[pl.ds(base, CROWS), :])
    pltpu.sync_copy(y2s, y2.at[pl.ds(base, CROWS), :])
    pltpu.sync_copy(lds, ld.at[pl.ds(base, CROWS)])


_combine_call = functools.partial(
    pl.kernel,
    mesh=plsc.VectorSubcoreMesh(core_axis_name="c", subcore_axis_name="s"),
    compiler_params=pltpu.CompilerParams(needs_layout_passes=False,
                                         use_tc_tiling_on_sc=False),
    out_type=[jax.ShapeDtypeStruct((NPAD, 2), jnp.float32),
              jax.ShapeDtypeStruct((NPAD, 2), jnp.float32),
              jax.ShapeDtypeStruct((NPAD,), jnp.float32)],
    scratch_types=[
        pltpu.VMEM((CCH, 16), jnp.float32),
        pltpu.VMEM((CCH, 16), jnp.float32),
        pltpu.VMEM((CCH, 4), jnp.float32),
        pltpu.VMEM((CROWS, 2), jnp.float32),
        pltpu.VMEM((CROWS, 2), jnp.float32),
        pltpu.VMEM((CROWS,), jnp.float32),
        pltpu.VMEM((16,), jnp.float32),
    ],
)(_combine_body)


def _build_M(F1_W, F1_a_src, F1_a_dst, F2_W, F2_a_src, F2_a_dst,
             G1_W, G1_a_src, G1_a_dst, G2_W, G2_a_src, G2_a_dst):
    z = jnp.zeros((2,), jnp.float32)
    cols = []
    cols.append(jnp.concatenate([F1_W @ F1_a_src, z]))
    cols.append(jnp.concatenate([F2_W @ F2_a_src, z]))
    cols.append(jnp.concatenate([z, G1_W @ G1_a_src]))
    cols.append(jnp.concatenate([z, G2_W @ G2_a_src]))
    cols.append(jnp.concatenate([F1_W @ F1_a_dst, z]))
    cols.append(jnp.concatenate([F2_W @ F2_a_dst, z]))
    cols.append(jnp.concatenate([z, G1_W @ G1_a_dst]))
    cols.append(jnp.concatenate([z, G2_W @ G2_a_dst]))
    cols.append(jnp.concatenate([F1_W[:, 0], z]))
    cols.append(jnp.concatenate([F1_W[:, 1], z]))
    cols.append(jnp.concatenate([F2_W[:, 0], z]))
    cols.append(jnp.concatenate([F2_W[:, 1], z]))
    cols.append(jnp.concatenate([z, G1_W[:, 0]]))
    cols.append(jnp.concatenate([z, G1_W[:, 1]]))
    cols.append(jnp.concatenate([z, G2_W[:, 0]]))
    cols.append(jnp.concatenate([z, G2_W[:, 1]]))
    return jnp.stack(cols, axis=1)


def kernel(x, edge_index,
           F1_W, F1_a_src, F1_a_dst, F1_b,
           F2_W, F2_a_src, F2_a_dst, F2_b,
           G1_W, G1_a_src, G1_a_dst, G1_b,
           G2_W, G2_a_src, G2_a_dst, G2_b):
    M = _build_M(F1_W, F1_a_src, F1_a_dst, F2_W, F2_a_src, F2_a_dst,
                 G1_W, G1_a_src, G1_a_dst, G2_W, G2_a_src, G2_a_dst)
    x_pad = jnp.pad(x, ((0, NPAD - N), (0, 0)))

    pad_e = jnp.full((EALLOC - E,), DUMMY, jnp.int32)
    src2 = jnp.concatenate([edge_index[0], pad_e]).reshape(EALLOC // 128, 128)
    dst2 = jnp.concatenate([edge_index[1], pad_e]).reshape(EALLOC // 128, 128)

    accf, _ = _edge_call(x_pad, M, src2, dst2)

    b8 = jnp.concatenate([F1_b, F2_b, G1_b, G2_b,
                          jnp.zeros((8,), jnp.float32)])
    y1, y2, ldp = _combine_call(accf, x_pad, b8)
    return (y1[:N], y2[:N], ldp[:N])
